# baseline (speedup 1.0000x reference)
import jax
import jax.numpy as jnp
from jax.experimental import pallas as pl
from jax.experimental.pallas import tpu as pltpu

T = 0.07
M = 0.5
K = 4096
BLKN = 2048


def _l2n(x):
    return x / jnp.sqrt(jnp.sum(x ** 2, axis=1, keepdims=True))


def _mm_kernel(n_total, out_ref, bank_ref, dps_ref):
    j = pl.program_id(0)
    dp = jax.lax.dot_general(out_ref[...], bank_ref[...],
                             (((1,), (1,)), ((), ())),
                             preferred_element_type=jnp.float32)
    col = j * BLKN + jax.lax.broadcasted_iota(jnp.int32, dp.shape, 1)
    dps_ref[...] = jnp.where(col < n_total, dp, -2.0)


def kernel(indices, outputs, gpu_idx, bank, cluster_labels):
    B, Dm = outputs.shape
    N = bank.shape[0]
    nb = (N + BLKN - 1) // BLKN
    npad = nb * BLKN

    out = _l2n(outputs)

    import functools
    dps = pl.pallas_call(
        functools.partial(_mm_kernel, N),
        grid=(nb,),
        in_specs=[
            pl.BlockSpec((B, Dm), lambda j: (0, 0)),
            pl.BlockSpec((BLKN, Dm), lambda j: (j, 0)),
        ],
        out_specs=pl.BlockSpec((B, BLKN), lambda j: (0, j)),
        out_shape=jax.ShapeDtypeStruct((B, npad), jnp.float32),
    )(out, bank)

    back_nei_dps, back_nei_idxs = jax.lax.top_k(dps, K)
    Z = 2876934.2 / 1281167 * N
    back_nei_probs = jnp.exp(back_nei_dps / T) / Z
    batch_labels = cluster_labels[:, indices]
    top_labels = jnp.take(cluster_labels, back_nei_idxs, axis=1)
    all_close_nei = jnp.any(batch_labels[:, :, None] == top_labels, axis=0)
    relative_probs = jnp.sum(jnp.where(all_close_nei, back_nei_probs,
                                       jnp.zeros_like(back_nei_probs)), axis=1)
    relative_probs = relative_probs / jnp.sum(back_nei_probs, axis=1)
    loss = -jnp.mean(jnp.log(relative_probs + 1e-07))[None]
    data_memory = jnp.take(bank, indices, axis=0)
    new_data_memory = _l2n(data_memory * M + (1.0 - M) * out)
    return (loss, new_data_memory)


# speedup vs baseline: 35.4379x; 35.4379x over previous
import functools

import jax
import jax.numpy as jnp
from jax.experimental import pallas as pl
from jax.experimental.pallas import tpu as pltpu

T = 0.07
M = 0.5
K = 4096
BLKN = 2048
P = 16


def _l2n(x):
    return x / jnp.sqrt(jnp.sum(x ** 2, axis=1, keepdims=True))


def _fold(x, width=128):
    parts = x.shape[1] // width
    t = x[:, :width]
    for s in range(1, parts):
        t = t + x[:, s * width:(s + 1) * width]
    return t


def _fused_kernel(nb, kk, out_ref, bank_ref, pen_ref, labels_ref, blab_ref,
                  s_ref, sc_ref, lo, hi, cl, ch, frac, acc, sacc, scacc):
    p = pl.program_id(0)
    j = pl.program_id(1)
    B = out_ref.shape[0]

    @pl.when((p == 0) & (j == 0))
    def _init():
        lo[...] = jnp.full((B, 1), -1.01, jnp.float32)
        hi[...] = jnp.full((B, 1), 1.01, jnp.float32)
        cl[...] = jnp.full((B, 1), 2 ** 30, jnp.int32)
        ch[...] = jnp.zeros((B, 1), jnp.int32)

    @pl.when((p > 0) & (j == 0))
    def _update():
        cnt = jnp.sum(acc[...], axis=1, keepdims=True)
        mid = (lo[...] + hi[...]) * 0.5
        take = cnt >= kk
        lo[...] = jnp.where(take, mid, lo[...])
        cl[...] = jnp.where(take, cnt, cl[...])
        hi[...] = jnp.where(take, hi[...], mid)
        ch[...] = jnp.where(take, ch[...], cnt)

    @pl.when(j == 0)
    def _reset():
        acc[...] = jnp.zeros_like(acc)

    @pl.when((p == P) & (j == 0))
    def _frac():
        nrem = (kk - ch[...]).astype(jnp.float32)
        nbr = jnp.maximum(cl[...] - ch[...], 1).astype(jnp.float32)
        frac[...] = jnp.clip(nrem / nbr, 0.0, 1.0)
        sacc[...] = jnp.zeros_like(sacc)
        scacc[...] = jnp.zeros_like(scacc)

    dp = jax.lax.dot_general(out_ref[...], bank_ref[...],
                             (((1,), (1,)), ((), ())),
                             preferred_element_type=jnp.float32)
    dp = dp + pen_ref[0]

    @pl.when(p < P)
    def _count():
        mid = (lo[...] + hi[...]) * 0.5
        c = (dp > mid).astype(jnp.int32)
        acc[...] = acc[...] + _fold(c)

    @pl.when(p == P)
    def _final():
        e = jnp.exp(dp * (1.0 / T))
        wgt = jnp.where(dp > hi[...], 1.0,
                        jnp.where(dp > lo[...], frac[...], 0.0))
        we = wgt * e
        lab = labels_ref[0]
        close = ((lab[0:1, :] == blab_ref[:, 0:1])
                 | (lab[1:2, :] == blab_ref[:, 1:2])
                 | (lab[2:3, :] == blab_ref[:, 2:3]))
        sacc[...] = sacc[...] + _fold(we)
        scacc[...] = scacc[...] + _fold(jnp.where(close, we, 0.0))

    @pl.when((p == P) & (j == nb - 1))
    def _writeout():
        s_ref[...] = jnp.broadcast_to(
            jnp.sum(sacc[...], axis=1, keepdims=True), s_ref.shape)
        sc_ref[...] = jnp.broadcast_to(
            jnp.sum(scacc[...], axis=1, keepdims=True), sc_ref.shape)


def kernel(indices, outputs, gpu_idx, bank, cluster_labels):
    B, Dm = outputs.shape
    N = bank.shape[0]
    nb = (N + BLKN - 1) // BLKN
    npad = nb * BLKN

    out = _l2n(outputs)
    out_bf = out.astype(jnp.bfloat16)
    bank_bf = jnp.pad(bank, ((0, npad - N), (0, 0))).astype(jnp.bfloat16)

    col = jnp.arange(npad, dtype=jnp.int32)
    pen = jnp.where(col < N, 0.0, -1e3).astype(jnp.float32)
    pen = pen.reshape(nb, 1, BLKN)

    labels = jnp.pad(cluster_labels, ((0, 0), (0, npad - N)),
                     constant_values=-1)
    labels = labels.reshape(cluster_labels.shape[0], nb, BLKN)
    labels = jnp.transpose(labels, (1, 0, 2))

    blab = jnp.pad(cluster_labels[:, indices].T, ((0, 0), (0, 5)),
                   constant_values=-2)

    grid = (P + 1, nb)
    s, sc = pl.pallas_call(
        functools.partial(_fused_kernel, nb, K),
        grid=grid,
        in_specs=[
            pl.BlockSpec((B, Dm), lambda p, j: (0, 0)),
            pl.BlockSpec((BLKN, Dm), lambda p, j: (j, 0)),
            pl.BlockSpec((1, 1, BLKN), lambda p, j: (j, 0, 0)),
            pl.BlockSpec((1, 3, BLKN), lambda p, j: (j, 0, 0)),
            pl.BlockSpec((B, 8), lambda p, j: (0, 0)),
        ],
        out_specs=[
            pl.BlockSpec((B, 128), lambda p, j: (0, 0)),
            pl.BlockSpec((B, 128), lambda p, j: (0, 0)),
        ],
        out_shape=[
            jax.ShapeDtypeStruct((B, 128), jnp.float32),
            jax.ShapeDtypeStruct((B, 128), jnp.float32),
        ],
        scratch_shapes=[
            pltpu.VMEM((B, 1), jnp.float32),
            pltpu.VMEM((B, 1), jnp.float32),
            pltpu.VMEM((B, 1), jnp.int32),
            pltpu.VMEM((B, 1), jnp.int32),
            pltpu.VMEM((B, 1), jnp.float32),
            pltpu.VMEM((B, 128), jnp.int32),
            pltpu.VMEM((B, 128), jnp.float32),
            pltpu.VMEM((B, 128), jnp.float32),
        ],
    )(out_bf, bank_bf, pen, labels, blab)

    s_all = s[:, 0]
    s_close = sc[:, 0]
    loss = -jnp.mean(jnp.log(s_close / s_all + 1e-07))[None]

    data_memory = jnp.take(bank, indices, axis=0)
    new_data_memory = _l2n(data_memory * M + (1.0 - M) * out)
    return (loss, new_data_memory)


# speedup vs baseline: 51.0695x; 1.4411x over previous
import functools

import jax
import jax.numpy as jnp
from jax.experimental import pallas as pl
from jax.experimental.pallas import tpu as pltpu

T = 0.07
M = 0.5
K = 4096
BLKN = 2048
P = 10


def _l2n(x):
    return x / jnp.sqrt(jnp.sum(x ** 2, axis=1, keepdims=True))


def _fold(x, width=128):
    parts = x.shape[1] // width
    t = x[:, :width]
    for s in range(1, parts):
        t = t + x[:, s * width:(s + 1) * width]
    return t


def _fused_kernel(nb, kk, out_ref, bank_ref, pen_ref, labels_ref, blab_ref,
                  s_ref, sc_ref, lo, hi, cl, ch, frac, acc, sacc, scacc):
    p = pl.program_id(0)
    j = pl.program_id(1)
    B = out_ref.shape[0]

    @pl.when((p == 0) & (j == 0))
    def _init():
        lo[...] = jnp.full((B, 1), -1.01, jnp.float32)
        hi[...] = jnp.full((B, 1), 1.01, jnp.float32)
        cl[...] = jnp.full((B, 1), 2 ** 30, jnp.int32)
        ch[...] = jnp.zeros((B, 1), jnp.int32)

    @pl.when((p > 0) & (j == 0))
    def _update():
        cnt = jnp.sum(acc[...], axis=1, keepdims=True)
        mid = (lo[...] + hi[...]) * 0.5
        take = cnt >= kk
        lo[...] = jnp.where(take, mid, lo[...])
        cl[...] = jnp.where(take, cnt, cl[...])
        hi[...] = jnp.where(take, hi[...], mid)
        ch[...] = jnp.where(take, ch[...], cnt)

    @pl.when(j == 0)
    def _reset():
        acc[...] = jnp.zeros_like(acc)

    @pl.when((p == P) & (j == 0))
    def _frac():
        nrem = (kk - ch[...]).astype(jnp.float32)
        nbr = jnp.maximum(cl[...] - ch[...], 1).astype(jnp.float32)
        frac[...] = jnp.clip(nrem / nbr, 0.0, 1.0)
        sacc[...] = jnp.zeros_like(sacc)
        scacc[...] = jnp.zeros_like(scacc)

    dp = jax.lax.dot_general(out_ref[...], bank_ref[...],
                             (((1,), (1,)), ((), ())),
                             preferred_element_type=jnp.float32)
    dp = dp + pen_ref[0]

    @pl.when(p < P)
    def _count():
        mid = (lo[...] + hi[...]) * 0.5
        c = (dp > mid).astype(jnp.int32)
        acc[...] = acc[...] + _fold(c)

    @pl.when(p == P)
    def _final():
        e = jnp.exp(dp * (1.0 / T))
        wgt = jnp.where(dp > hi[...], 1.0,
                        jnp.where(dp > lo[...], frac[...], 0.0))
        we = wgt * e
        lab = labels_ref[0]
        close = ((lab[0:1, :] == blab_ref[:, 0:1])
                 | (lab[1:2, :] == blab_ref[:, 1:2])
                 | (lab[2:3, :] == blab_ref[:, 2:3]))
        sacc[...] = sacc[...] + _fold(we)
        scacc[...] = scacc[...] + _fold(jnp.where(close, we, 0.0))

    @pl.when((p == P) & (j == nb - 1))
    def _writeout():
        s_ref[...] = jnp.broadcast_to(
            jnp.sum(sacc[...], axis=1, keepdims=True), s_ref.shape)
        sc_ref[...] = jnp.broadcast_to(
            jnp.sum(scacc[...], axis=1, keepdims=True), sc_ref.shape)


def kernel(indices, outputs, gpu_idx, bank, cluster_labels):
    B, Dm = outputs.shape
    N = bank.shape[0]
    nb = (N + BLKN - 1) // BLKN
    npad = nb * BLKN

    out = _l2n(outputs)
    out_bf = out.astype(jnp.bfloat16)
    bank_bf = jnp.pad(bank, ((0, npad - N), (0, 0))).astype(jnp.bfloat16)

    col = jnp.arange(npad, dtype=jnp.int32)
    pen = jnp.where(col < N, 0.0, -1e3).astype(jnp.float32)
    pen = pen.reshape(nb, 1, BLKN)

    labels = jnp.pad(cluster_labels, ((0, 0), (0, npad - N)),
                     constant_values=-1)
    labels = labels.reshape(cluster_labels.shape[0], nb, BLKN)
    labels = jnp.transpose(labels, (1, 0, 2))

    blab = jnp.pad(cluster_labels[:, indices].T, ((0, 0), (0, 5)),
                   constant_values=-2)

    grid = (P + 1, nb)
    s, sc = pl.pallas_call(
        functools.partial(_fused_kernel, nb, K),
        grid=grid,
        in_specs=[
            pl.BlockSpec((B, Dm), lambda p, j: (0, 0)),
            pl.BlockSpec((BLKN, Dm), lambda p, j: (j, 0)),
            pl.BlockSpec((1, 1, BLKN), lambda p, j: (j, 0, 0)),
            pl.BlockSpec((1, 3, BLKN), lambda p, j: (j, 0, 0)),
            pl.BlockSpec((B, 8), lambda p, j: (0, 0)),
        ],
        out_specs=[
            pl.BlockSpec((B, 128), lambda p, j: (0, 0)),
            pl.BlockSpec((B, 128), lambda p, j: (0, 0)),
        ],
        out_shape=[
            jax.ShapeDtypeStruct((B, 128), jnp.float32),
            jax.ShapeDtypeStruct((B, 128), jnp.float32),
        ],
        scratch_shapes=[
            pltpu.VMEM((B, 1), jnp.float32),
            pltpu.VMEM((B, 1), jnp.float32),
            pltpu.VMEM((B, 1), jnp.int32),
            pltpu.VMEM((B, 1), jnp.int32),
            pltpu.VMEM((B, 1), jnp.float32),
            pltpu.VMEM((B, 128), jnp.int32),
            pltpu.VMEM((B, 128), jnp.float32),
            pltpu.VMEM((B, 128), jnp.float32),
        ],
    )(out_bf, bank_bf, pen, labels, blab)

    s_all = s[:, 0]
    s_close = sc[:, 0]
    loss = -jnp.mean(jnp.log(s_close / s_all + 1e-07))[None]

    data_memory = jnp.take(bank, indices, axis=0)
    new_data_memory = _l2n(data_memory * M + (1.0 - M) * out)
    return (loss, new_data_memory)


# speedup vs baseline: 56.0519x; 1.0976x over previous
import functools

import jax
import jax.numpy as jnp
from jax.experimental import pallas as pl
from jax.experimental.pallas import tpu as pltpu

T = 0.07
M = 0.5
K = 4096
BLKN = 2048
P = 6
E_LO = 0.12
E_HI = 0.19


def _l2n(x):
    return x / jnp.sqrt(jnp.sum(x ** 2, axis=1, keepdims=True))


def _colsum(x):
    ones = jnp.ones((x.shape[1], 8), jnp.float32)
    r = jax.lax.dot_general(x, ones, (((1,), (0,)), ((), ())),
                            preferred_element_type=jnp.float32)
    return r[:, 0:1]


def _fused_kernel(nb, kk, nreal, out_ref, bank_ref, pen_ref, labels_ref,
                  blab_ref, s_ref, sc_ref,
                  lo, hi, cl, ch, frac, acc, acc2, sacc, scacc):
    p = pl.program_id(0)
    j = pl.program_id(1)
    B = out_ref.shape[0]
    kf = jnp.float32(kk)

    @pl.when((p == 0) & (j == 0))
    def _init0():
        acc[...] = jnp.zeros_like(acc)
        acc2[...] = jnp.zeros_like(acc2)

    @pl.when((p == 1) & (j == 0))
    def _warm():
        clo = acc[...]
        chi = acc2[...]
        oklo = clo >= kf
        okhi = chi < kf
        lo[...] = jnp.where(oklo, E_LO, -1.01)
        cl[...] = jnp.where(oklo, clo, jnp.float32(nreal))
        hi[...] = jnp.where(okhi, E_HI, 1.01)
        ch[...] = jnp.where(okhi, chi, 0.0)
        acc[...] = jnp.zeros_like(acc)

    @pl.when((p > 1) & (j == 0))
    def _update():
        cnt = acc[...]
        mid = (lo[...] + hi[...]) * 0.5
        take = cnt >= kf
        lo[...] = jnp.where(take, mid, lo[...])
        cl[...] = jnp.where(take, cnt, cl[...])
        hi[...] = jnp.where(take, hi[...], mid)
        ch[...] = jnp.where(take, ch[...], cnt)
        acc[...] = jnp.zeros_like(acc)

    @pl.when((p == P + 1) & (j == 0))
    def _frac():
        nrem = kf - ch[...]
        nbr = jnp.maximum(cl[...] - ch[...], 1.0)
        frac[...] = jnp.clip(nrem / nbr, 0.0, 1.0)
        sacc[...] = jnp.zeros_like(sacc)
        scacc[...] = jnp.zeros_like(scacc)

    dp = jax.lax.dot_general(out_ref[...], bank_ref[...],
                             (((1,), (1,)), ((), ())),
                             preferred_element_type=jnp.float32)
    dp = dp + pen_ref[0]

    @pl.when(p == 0)
    def _count0():
        acc[...] = acc[...] + _colsum(jnp.where(dp > E_LO, 1.0, 0.0))
        acc2[...] = acc2[...] + _colsum(jnp.where(dp > E_HI, 1.0, 0.0))

    @pl.when((p > 0) & (p <= P))
    def _count():
        mid = (lo[...] + hi[...]) * 0.5
        acc[...] = acc[...] + _colsum(jnp.where(dp > mid, 1.0, 0.0))

    @pl.when(p == P + 1)
    def _final():
        e = jnp.exp(dp * (1.0 / T))
        wgt = jnp.where(dp > hi[...], 1.0,
                        jnp.where(dp > lo[...], frac[...], 0.0))
        we = wgt * e
        lab = labels_ref[0]
        close = ((lab[0:1, :] == blab_ref[:, 0:1])
                 | (lab[1:2, :] == blab_ref[:, 1:2])
                 | (lab[2:3, :] == blab_ref[:, 2:3]))
        sacc[...] = sacc[...] + _colsum(we)
        scacc[...] = scacc[...] + _colsum(jnp.where(close, we, 0.0))

    @pl.when((p == P + 1) & (j == nb - 1))
    def _writeout():
        s_ref[...] = jnp.broadcast_to(sacc[...], s_ref.shape)
        sc_ref[...] = jnp.broadcast_to(scacc[...], sc_ref.shape)


def kernel(indices, outputs, gpu_idx, bank, cluster_labels):
    B, Dm = outputs.shape
    N = bank.shape[0]
    nb = (N + BLKN - 1) // BLKN
    npad = nb * BLKN

    out = _l2n(outputs)
    out_bf = out.astype(jnp.bfloat16)
    bank_bf = jnp.pad(bank, ((0, npad - N), (0, 0))).astype(jnp.bfloat16)

    col = jnp.arange(npad, dtype=jnp.int32)
    pen = jnp.where(col < N, 0.0, -1e3).astype(jnp.float32)
    pen = pen.reshape(nb, 1, BLKN)

    labels = jnp.pad(cluster_labels, ((0, 0), (0, npad - N)),
                     constant_values=-1)
    labels = labels.reshape(cluster_labels.shape[0], nb, BLKN)
    labels = jnp.transpose(labels, (1, 0, 2))

    blab = jnp.pad(cluster_labels[:, indices].T, ((0, 0), (0, 5)),
                   constant_values=-2)

    grid = (P + 2, nb)
    s, sc = pl.pallas_call(
        functools.partial(_fused_kernel, nb, K, N),
        grid=grid,
        in_specs=[
            pl.BlockSpec((B, Dm), lambda p, j: (0, 0)),
            pl.BlockSpec((BLKN, Dm), lambda p, j: (j, 0)),
            pl.BlockSpec((1, 1, BLKN), lambda p, j: (j, 0, 0)),
            pl.BlockSpec((1, 3, BLKN), lambda p, j: (j, 0, 0)),
            pl.BlockSpec((B, 8), lambda p, j: (0, 0)),
        ],
        out_specs=[
            pl.BlockSpec((B, 128), lambda p, j: (0, 0)),
            pl.BlockSpec((B, 128), lambda p, j: (0, 0)),
        ],
        out_shape=[
            jax.ShapeDtypeStruct((B, 128), jnp.float32),
            jax.ShapeDtypeStruct((B, 128), jnp.float32),
        ],
        scratch_shapes=[
            pltpu.VMEM((B, 1), jnp.float32),
            pltpu.VMEM((B, 1), jnp.float32),
            pltpu.VMEM((B, 1), jnp.float32),
            pltpu.VMEM((B, 1), jnp.float32),
            pltpu.VMEM((B, 1), jnp.float32),
            pltpu.VMEM((B, 1), jnp.float32),
            pltpu.VMEM((B, 1), jnp.float32),
            pltpu.VMEM((B, 1), jnp.float32),
            pltpu.VMEM((B, 1), jnp.float32),
        ],
    )(out_bf, bank_bf, pen, labels, blab)

    s_all = s[:, 0]
    s_close = sc[:, 0]
    loss = -jnp.mean(jnp.log(s_close / s_all + 1e-07))[None]

    data_memory = jnp.take(bank, indices, axis=0)
    new_data_memory = _l2n(data_memory * M + (1.0 - M) * out)
    return (loss, new_data_memory)


# speedup vs baseline: 67.4360x; 1.2031x over previous
import functools

import jax
import jax.numpy as jnp
from jax.experimental import pallas as pl
from jax.experimental.pallas import tpu as pltpu

T = 0.07
M = 0.5
K = 4096
BLKN = 2048
P = 5
E_LO = 0.12
E_HI = 0.19


def _l2n(x):
    return x / jnp.sqrt(jnp.sum(x ** 2, axis=1, keepdims=True))


def _colsum(x, width=128):
    parts = x.shape[1] // width
    t = x[:, :width]
    for s in range(1, parts):
        t = t + x[:, s * width:(s + 1) * width]
    return jnp.sum(t, axis=1, keepdims=True)


def _fused_kernel(nb, kk, nreal, out_ref, bank_ref, pen_ref, labels_ref,
                  blab_ref, s_ref, sc_ref,
                  lo, hi, cl, ch, frac, acc, acc2, sacc, scacc):
    p = pl.program_id(0)
    j = pl.program_id(1)
    B = out_ref.shape[0]
    kf = jnp.float32(kk)

    @pl.when((p == 0) & (j == 0))
    def _init0():
        acc[...] = jnp.zeros_like(acc)
        acc2[...] = jnp.zeros_like(acc2)

    @pl.when((p == 1) & (j == 0))
    def _warm():
        clo = acc[...]
        chi = acc2[...]
        oklo = clo >= kf
        okhi = chi < kf
        lo[...] = jnp.where(oklo, E_LO, -1.01)
        cl[...] = jnp.where(oklo, clo, jnp.float32(nreal))
        hi[...] = jnp.where(okhi, E_HI, 1.01)
        ch[...] = jnp.where(okhi, chi, 0.0)
        acc[...] = jnp.zeros_like(acc)

    @pl.when((p > 1) & (j == 0))
    def _update():
        cnt = acc[...]
        mid = (lo[...] + hi[...]) * 0.5
        take = cnt >= kf
        lo[...] = jnp.where(take, mid, lo[...])
        cl[...] = jnp.where(take, cnt, cl[...])
        hi[...] = jnp.where(take, hi[...], mid)
        ch[...] = jnp.where(take, ch[...], cnt)
        acc[...] = jnp.zeros_like(acc)

    @pl.when((p == P + 1) & (j == 0))
    def _frac():
        nrem = kf - ch[...]
        nbr = jnp.maximum(cl[...] - ch[...], 1.0)
        frac[...] = jnp.clip(nrem / nbr, 0.0, 1.0)
        sacc[...] = jnp.zeros_like(sacc)
        scacc[...] = jnp.zeros_like(scacc)

    dp = jax.lax.dot_general(out_ref[...], bank_ref[...],
                             (((1,), (1,)), ((), ())),
                             preferred_element_type=jnp.float32)
    dp = dp + pen_ref[0]

    @pl.when(p == 0)
    def _count0():
        acc[...] = acc[...] + _colsum(jnp.where(dp > E_LO, 1.0, 0.0))
        acc2[...] = acc2[...] + _colsum(jnp.where(dp > E_HI, 1.0, 0.0))

    @pl.when((p > 0) & (p <= P))
    def _count():
        mid = (lo[...] + hi[...]) * 0.5
        acc[...] = acc[...] + _colsum(jnp.where(dp > mid, 1.0, 0.0))

    @pl.when(p == P + 1)
    def _final():
        e = jnp.exp(dp * (1.0 / T))
        wgt = jnp.where(dp > hi[...], 1.0,
                        jnp.where(dp > lo[...], frac[...], 0.0))
        we = wgt * e
        lab = labels_ref[0]
        close = ((lab[0:1, :] == blab_ref[:, 0:1])
                 | (lab[1:2, :] == blab_ref[:, 1:2])
                 | (lab[2:3, :] == blab_ref[:, 2:3]))
        sacc[...] = sacc[...] + _colsum(we)
        scacc[...] = scacc[...] + _colsum(jnp.where(close, we, 0.0))

    @pl.when((p == P + 1) & (j == nb - 1))
    def _writeout():
        s_ref[...] = jnp.broadcast_to(sacc[...], s_ref.shape)
        sc_ref[...] = jnp.broadcast_to(scacc[...], sc_ref.shape)


def kernel(indices, outputs, gpu_idx, bank, cluster_labels):
    B, Dm = outputs.shape
    N = bank.shape[0]
    nb = (N + BLKN - 1) // BLKN
    npad = nb * BLKN

    out = _l2n(outputs)
    out_bf = out.astype(jnp.bfloat16)
    bank_bf = jnp.pad(bank, ((0, npad - N), (0, 0))).astype(jnp.bfloat16)

    col = jnp.arange(npad, dtype=jnp.int32)
    pen = jnp.where(col < N, 0.0, -1e3).astype(jnp.float32)
    pen = pen.reshape(nb, 1, BLKN)

    labels = jnp.pad(cluster_labels, ((0, 0), (0, npad - N)),
                     constant_values=-1)
    labels = labels.reshape(cluster_labels.shape[0], nb, BLKN)
    labels = jnp.transpose(labels, (1, 0, 2))

    blab = jnp.pad(cluster_labels[:, indices].T, ((0, 0), (0, 5)),
                   constant_values=-2)

    grid = (P + 2, nb)
    s, sc = pl.pallas_call(
        functools.partial(_fused_kernel, nb, K, N),
        grid=grid,
        in_specs=[
            pl.BlockSpec((B, Dm), lambda p, j: (0, 0)),
            pl.BlockSpec((BLKN, Dm), lambda p, j: (j, 0)),
            pl.BlockSpec((1, 1, BLKN), lambda p, j: (j, 0, 0)),
            pl.BlockSpec((1, 3, BLKN), lambda p, j: (j, 0, 0)),
            pl.BlockSpec((B, 8), lambda p, j: (0, 0)),
        ],
        out_specs=[
            pl.BlockSpec((B, 128), lambda p, j: (0, 0)),
            pl.BlockSpec((B, 128), lambda p, j: (0, 0)),
        ],
        out_shape=[
            jax.ShapeDtypeStruct((B, 128), jnp.float32),
            jax.ShapeDtypeStruct((B, 128), jnp.float32),
        ],
        scratch_shapes=[
            pltpu.VMEM((B, 1), jnp.float32),
            pltpu.VMEM((B, 1), jnp.float32),
            pltpu.VMEM((B, 1), jnp.float32),
            pltpu.VMEM((B, 1), jnp.float32),
            pltpu.VMEM((B, 1), jnp.float32),
            pltpu.VMEM((B, 1), jnp.float32),
            pltpu.VMEM((B, 1), jnp.float32),
            pltpu.VMEM((B, 1), jnp.float32),
            pltpu.VMEM((B, 1), jnp.float32),
        ],
    )(out_bf, bank_bf, pen, labels, blab)

    s_all = s[:, 0]
    s_close = sc[:, 0]
    loss = -jnp.mean(jnp.log(s_close / s_all + 1e-07))[None]

    data_memory = jnp.take(bank, indices, axis=0)
    new_data_memory = _l2n(data_memory * M + (1.0 - M) * out)
    return (loss, new_data_memory)


# speedup vs baseline: 70.2219x; 1.0413x over previous
import functools

import jax
import jax.numpy as jnp
from jax import lax
from jax.experimental import pallas as pl
from jax.experimental.pallas import tpu as pltpu
from jax.experimental.pallas import tpu_sc as plsc

T = 0.07
M = 0.5
K = 4096
BLKN = 2048
P = 5
E_LO = 0.12
E_HI = 0.19


def _l2n(x):
    return x / jnp.sqrt(jnp.sum(x ** 2, axis=1, keepdims=True))


def _colsum(x, width=128):
    parts = x.shape[1] // width
    t = x[:, :width]
    for s in range(1, parts):
        t = t + x[:, s * width:(s + 1) * width]
    return jnp.sum(t, axis=1, keepdims=True)


def _gather_sc(bank, lab0, lab1, lab2, indices):
    B = indices.shape[0]
    D = bank.shape[1]
    NW = 32
    bpw = B // NW
    mesh = plsc.VectorSubcoreMesh(core_axis_name="c", subcore_axis_name="s")

    @functools.partial(
        pl.kernel, mesh=mesh,
        out_type=[jax.ShapeDtypeStruct((B, D), jnp.float32),
                  jax.ShapeDtypeStruct((B,), jnp.int32),
                  jax.ShapeDtypeStruct((B,), jnp.int32),
                  jax.ShapeDtypeStruct((B,), jnp.int32)],
        scratch_types=[pltpu.VMEM((bpw,), jnp.int32),
                       pltpu.VMEM((bpw, D), jnp.float32),
                       pltpu.VMEM((bpw,), jnp.int32),
                       pltpu.SemaphoreType.DMA])
    def k(bank_hbm, l0, l1, l2, idx_hbm, rows_out, b0, b1, b2,
          idx_v, rows_v, lab_v, sem):
        wid = lax.axis_index("s") * 2 + lax.axis_index("c")
        base = wid * bpw
        pltpu.sync_copy(idx_hbm.at[pl.ds(base, bpw)], idx_v)
        pltpu.async_copy(bank_hbm.at[idx_v], rows_v, sem).wait()
        pltpu.sync_copy(rows_v, rows_out.at[pl.ds(base, bpw)])
        for lm, bm in ((l0, b0), (l1, b1), (l2, b2)):
            pltpu.async_copy(lm.at[idx_v], lab_v, sem).wait()
            pltpu.sync_copy(lab_v, bm.at[pl.ds(base, bpw)])

    return k(bank, lab0, lab1, lab2, indices)


def _fused_kernel(nb, kk, padn, nreal, out_ref, bank_ref, labels_ref,
                  blab_ref, s_ref, sc_ref,
                  lo, hi, cl, ch, frac, acc, acc2, sacc, scacc):
    p = pl.program_id(0)
    j = pl.program_id(1)
    B = out_ref.shape[0]
    kf = jnp.float32(kk)
    padf = jnp.float32(padn)

    @pl.when((p == 0) & (j == 0))
    def _init0():
        acc[...] = jnp.zeros_like(acc)
        acc2[...] = jnp.zeros_like(acc2)

    @pl.when((p == 1) & (j == 0))
    def _warm():
        clo = acc[...]
        chi = acc2[...]
        oklo = clo >= kf
        okhi = chi < kf
        lo[...] = jnp.where(oklo, E_LO, -1.01)
        cl[...] = jnp.where(oklo, clo, jnp.float32(nreal))
        hi[...] = jnp.where(okhi, E_HI, 1.01)
        ch[...] = jnp.where(okhi, chi, 0.0)
        acc[...] = jnp.zeros_like(acc)

    @pl.when((p > 1) & (j == 0))
    def _update():
        mid = (lo[...] + hi[...]) * 0.5
        cnt = acc[...] - jnp.where(mid < 0.0, padf, 0.0)
        take = cnt >= kf
        lo[...] = jnp.where(take, mid, lo[...])
        cl[...] = jnp.where(take, cnt, cl[...])
        hi[...] = jnp.where(take, hi[...], mid)
        ch[...] = jnp.where(take, ch[...], cnt)
        acc[...] = jnp.zeros_like(acc)

    @pl.when((p == P + 1) & (j == 0))
    def _frac():
        nrem = kf - ch[...]
        nbr = jnp.maximum(cl[...] - ch[...], 1.0)
        frac[...] = jnp.clip(nrem / nbr, 0.0, 1.0)
        sacc[...] = jnp.zeros_like(sacc)
        scacc[...] = jnp.zeros_like(scacc)

    dp = jax.lax.dot_general(out_ref[...], bank_ref[...],
                             (((1,), (1,)), ((), ())),
                             preferred_element_type=jnp.float32)

    @pl.when(p == 0)
    def _count0():
        acc[...] = acc[...] + _colsum(jnp.where(dp > E_LO, 1.0, 0.0))
        acc2[...] = acc2[...] + _colsum(jnp.where(dp > E_HI, 1.0, 0.0))

    @pl.when((p > 0) & (p <= P))
    def _count():
        mid = (lo[...] + hi[...]) * 0.5
        acc[...] = acc[...] + _colsum(jnp.where(dp > mid, 1.0, 0.0))

    @pl.when(p == P + 1)
    def _final():
        e = jnp.exp(dp * (1.0 / T))
        wgt = jnp.where(dp > hi[...], 1.0,
                        jnp.where(dp > lo[...], frac[...], 0.0))
        we = wgt * e
        lab = labels_ref[0]
        close = ((lab[0:1, :] == blab_ref[:, 0:1])
                 | (lab[1:2, :] == blab_ref[:, 1:2])
                 | (lab[2:3, :] == blab_ref[:, 2:3]))
        sacc[...] = sacc[...] + _colsum(we)
        scacc[...] = scacc[...] + _colsum(jnp.where(close, we, 0.0))

    @pl.when((p == P + 1) & (j == nb - 1))
    def _writeout():
        wgt0 = jnp.where(hi[...] < 0.0, 1.0,
                         jnp.where(lo[...] < 0.0, frac[...], 0.0))
        s_ref[...] = jnp.broadcast_to(sacc[...] - padf * wgt0, s_ref.shape)
        sc_ref[...] = jnp.broadcast_to(scacc[...], sc_ref.shape)


def kernel(indices, outputs, gpu_idx, bank, cluster_labels):
    B, Dm = outputs.shape
    N = bank.shape[0]
    nb = (N + BLKN - 1) // BLKN
    npad = nb * BLKN

    out = _l2n(outputs)
    out_bf = out.astype(jnp.bfloat16)
    bank_bf = jnp.pad(bank, ((0, npad - N), (0, 0))).astype(jnp.bfloat16)

    labels = jnp.pad(cluster_labels, ((0, 0), (0, npad - N)),
                     constant_values=-1)
    labels = labels.reshape(cluster_labels.shape[0], nb, BLKN)
    labels = jnp.transpose(labels, (1, 0, 2))

    rows, b0, b1, b2 = _gather_sc(bank, cluster_labels[0],
                                  cluster_labels[1], cluster_labels[2],
                                  indices)
    blab = jnp.pad(jnp.stack([b0, b1, b2], axis=1), ((0, 0), (0, 5)),
                   constant_values=-2)

    grid = (P + 2, nb)
    s, sc = pl.pallas_call(
        functools.partial(_fused_kernel, nb, K, npad - N, N),
        grid=grid,
        in_specs=[
            pl.BlockSpec((B, Dm), lambda p, j: (0, 0)),
            pl.BlockSpec((BLKN, Dm), lambda p, j: (j, 0)),
            pl.BlockSpec((1, 3, BLKN), lambda p, j: (j, 0, 0)),
            pl.BlockSpec((B, 8), lambda p, j: (0, 0)),
        ],
        out_specs=[
            pl.BlockSpec((B, 128), lambda p, j: (0, 0)),
            pl.BlockSpec((B, 128), lambda p, j: (0, 0)),
        ],
        out_shape=[
            jax.ShapeDtypeStruct((B, 128), jnp.float32),
            jax.ShapeDtypeStruct((B, 128), jnp.float32),
        ],
        scratch_shapes=[
            pltpu.VMEM((B, 1), jnp.float32),
            pltpu.VMEM((B, 1), jnp.float32),
            pltpu.VMEM((B, 1), jnp.float32),
            pltpu.VMEM((B, 1), jnp.float32),
            pltpu.VMEM((B, 1), jnp.float32),
            pltpu.VMEM((B, 1), jnp.float32),
            pltpu.VMEM((B, 1), jnp.float32),
            pltpu.VMEM((B, 1), jnp.float32),
            pltpu.VMEM((B, 1), jnp.float32),
        ],
    )(out_bf, bank_bf, labels, blab)

    s_all = s[:, 0]
    s_close = sc[:, 0]
    loss = -jnp.mean(jnp.log(s_close / s_all + 1e-07))[None]

    new_data_memory = _l2n(rows * M + (1.0 - M) * out)
    return (loss, new_data_memory)


# speedup vs baseline: 74.4012x; 1.0595x over previous
import functools

import jax
import jax.numpy as jnp
from jax import lax
from jax.experimental import pallas as pl
from jax.experimental.pallas import tpu as pltpu
from jax.experimental.pallas import tpu_sc as plsc

T = 0.07
M = 0.5
K = 4096
BLKN = 4096
P = 5
E_LO = 0.12
E_HI = 0.19


def _l2n(x):
    return x / jnp.sqrt(jnp.sum(x ** 2, axis=1, keepdims=True))


def _colsum(x, width=128):
    parts = x.shape[1] // width
    t = x[:, :width]
    for s in range(1, parts):
        t = t + x[:, s * width:(s + 1) * width]
    return jnp.sum(t, axis=1, keepdims=True)


def _gather_sc(bank, lab0, lab1, lab2, indices):
    B = indices.shape[0]
    D = bank.shape[1]
    NW = 32
    bpw = B // NW
    mesh = plsc.VectorSubcoreMesh(core_axis_name="c", subcore_axis_name="s")

    @functools.partial(
        pl.kernel, mesh=mesh,
        out_type=[jax.ShapeDtypeStruct((B, D), jnp.float32),
                  jax.ShapeDtypeStruct((B,), jnp.int32),
                  jax.ShapeDtypeStruct((B,), jnp.int32),
                  jax.ShapeDtypeStruct((B,), jnp.int32)],
        scratch_types=[pltpu.VMEM((bpw,), jnp.int32),
                       pltpu.VMEM((bpw, D), jnp.float32),
                       pltpu.VMEM((bpw,), jnp.int32),
                       pltpu.SemaphoreType.DMA])
    def k(bank_hbm, l0, l1, l2, idx_hbm, rows_out, b0, b1, b2,
          idx_v, rows_v, lab_v, sem):
        wid = lax.axis_index("s") * 2 + lax.axis_index("c")
        base = wid * bpw
        pltpu.sync_copy(idx_hbm.at[pl.ds(base, bpw)], idx_v)
        pltpu.async_copy(bank_hbm.at[idx_v], rows_v, sem).wait()
        pltpu.sync_copy(rows_v, rows_out.at[pl.ds(base, bpw)])
        for lm, bm in ((l0, b0), (l1, b1), (l2, b2)):
            pltpu.async_copy(lm.at[idx_v], lab_v, sem).wait()
            pltpu.sync_copy(lab_v, bm.at[pl.ds(base, bpw)])

    return k(bank, lab0, lab1, lab2, indices)


def _fused_kernel(nb, kk, padn, nreal, out_ref, bank_ref, labels_ref,
                  blab_ref, s_ref, sc_ref,
                  lo, hi, cl, ch, frac, acc, acc2, sacc, scacc):
    p = pl.program_id(0)
    j = pl.program_id(1)
    B = out_ref.shape[0]
    kf = jnp.float32(kk)
    padf = jnp.float32(padn)

    @pl.when((p == 0) & (j == 0))
    def _init0():
        acc[...] = jnp.zeros_like(acc)
        acc2[...] = jnp.zeros_like(acc2)

    @pl.when((p == 1) & (j == 0))
    def _warm():
        clo = acc[...]
        chi = acc2[...]
        oklo = clo >= kf
        okhi = chi < kf
        lo[...] = jnp.where(oklo, E_LO, -1.01)
        cl[...] = jnp.where(oklo, clo, jnp.float32(nreal))
        hi[...] = jnp.where(okhi, E_HI, 1.01)
        ch[...] = jnp.where(okhi, chi, 0.0)
        acc[...] = jnp.zeros_like(acc)

    @pl.when((p > 1) & (j == 0))
    def _update():
        mid = (lo[...] + hi[...]) * 0.5
        cnt = acc[...] - jnp.where(mid < 0.0, padf, 0.0)
        take = cnt >= kf
        lo[...] = jnp.where(take, mid, lo[...])
        cl[...] = jnp.where(take, cnt, cl[...])
        hi[...] = jnp.where(take, hi[...], mid)
        ch[...] = jnp.where(take, ch[...], cnt)
        acc[...] = jnp.zeros_like(acc)

    @pl.when((p == P + 1) & (j == 0))
    def _frac():
        nrem = kf - ch[...]
        nbr = jnp.maximum(cl[...] - ch[...], 1.0)
        frac[...] = jnp.clip(nrem / nbr, 0.0, 1.0)
        sacc[...] = jnp.zeros_like(sacc)
        scacc[...] = jnp.zeros_like(scacc)

    dp = jax.lax.dot_general(out_ref[...], bank_ref[...],
                             (((1,), (1,)), ((), ())),
                             preferred_element_type=jnp.float32)

    @pl.when(p == 0)
    def _count0():
        acc[...] = acc[...] + _colsum(jnp.where(dp > E_LO, 1.0, 0.0))
        acc2[...] = acc2[...] + _colsum(jnp.where(dp > E_HI, 1.0, 0.0))

    @pl.when((p > 0) & (p <= P))
    def _count():
        mid = (lo[...] + hi[...]) * 0.5
        acc[...] = acc[...] + _colsum(jnp.where(dp > mid, 1.0, 0.0))

    @pl.when(p == P + 1)
    def _final():
        e = jnp.exp(dp * (1.0 / T))
        wgt = jnp.where(dp > hi[...], 1.0,
                        jnp.where(dp > lo[...], frac[...], 0.0))
        we = wgt * e
        lab = labels_ref[0]
        close = ((lab[0:1, :] == blab_ref[:, 0:1])
                 | (lab[1:2, :] == blab_ref[:, 1:2])
                 | (lab[2:3, :] == blab_ref[:, 2:3]))
        sacc[...] = sacc[...] + _colsum(we)
        scacc[...] = scacc[...] + _colsum(jnp.where(close, we, 0.0))

    @pl.when((p == P + 1) & (j == nb - 1))
    def _writeout():
        wgt0 = jnp.where(hi[...] < 0.0, 1.0,
                         jnp.where(lo[...] < 0.0, frac[...], 0.0))
        s_ref[...] = jnp.broadcast_to(sacc[...] - padf * wgt0, s_ref.shape)
        sc_ref[...] = jnp.broadcast_to(scacc[...], sc_ref.shape)


def kernel(indices, outputs, gpu_idx, bank, cluster_labels):
    B, Dm = outputs.shape
    N = bank.shape[0]
    nb = (N + BLKN - 1) // BLKN
    npad = nb * BLKN

    out = _l2n(outputs)
    out_bf = out.astype(jnp.bfloat16)
    bank_bf = jnp.pad(bank, ((0, npad - N), (0, 0))).astype(jnp.bfloat16)

    labels = jnp.pad(cluster_labels, ((0, 0), (0, npad - N)),
                     constant_values=-1)
    labels = labels.reshape(cluster_labels.shape[0], nb, BLKN)
    labels = jnp.transpose(labels, (1, 0, 2))

    rows, b0, b1, b2 = _gather_sc(bank, cluster_labels[0],
                                  cluster_labels[1], cluster_labels[2],
                                  indices)
    blab = jnp.pad(jnp.stack([b0, b1, b2], axis=1), ((0, 0), (0, 5)),
                   constant_values=-2)

    grid = (P + 2, nb)
    s, sc = pl.pallas_call(
        functools.partial(_fused_kernel, nb, K, npad - N, N),
        grid=grid,
        in_specs=[
            pl.BlockSpec((B, Dm), lambda p, j: (0, 0)),
            pl.BlockSpec((BLKN, Dm), lambda p, j: (j, 0)),
            pl.BlockSpec((1, 3, BLKN), lambda p, j: (j, 0, 0)),
            pl.BlockSpec((B, 8), lambda p, j: (0, 0)),
        ],
        out_specs=[
            pl.BlockSpec((B, 128), lambda p, j: (0, 0)),
            pl.BlockSpec((B, 128), lambda p, j: (0, 0)),
        ],
        out_shape=[
            jax.ShapeDtypeStruct((B, 128), jnp.float32),
            jax.ShapeDtypeStruct((B, 128), jnp.float32),
        ],
        scratch_shapes=[
            pltpu.VMEM((B, 1), jnp.float32),
            pltpu.VMEM((B, 1), jnp.float32),
            pltpu.VMEM((B, 1), jnp.float32),
            pltpu.VMEM((B, 1), jnp.float32),
            pltpu.VMEM((B, 1), jnp.float32),
            pltpu.VMEM((B, 1), jnp.float32),
            pltpu.VMEM((B, 1), jnp.float32),
            pltpu.VMEM((B, 1), jnp.float32),
            pltpu.VMEM((B, 1), jnp.float32),
        ],
    )(out_bf, bank_bf, labels, blab)

    s_all = s[:, 0]
    s_close = sc[:, 0]
    loss = -jnp.mean(jnp.log(s_close / s_all + 1e-07))[None]

    new_data_memory = _l2n(rows * M + (1.0 - M) * out)
    return (loss, new_data_memory)


# speedup vs baseline: 82.4780x; 1.1086x over previous
import functools

import jax
import jax.numpy as jnp
from jax import lax
from jax.experimental import pallas as pl
from jax.experimental.pallas import tpu as pltpu
from jax.experimental.pallas import tpu_sc as plsc

T = 0.07
M = 0.5
K = 4096
BLKN = 4096
P = 4
E_LO = 0.12
E_HI = 0.19


def _l2n(x):
    return x / jnp.sqrt(jnp.sum(x ** 2, axis=1, keepdims=True))


def _colsum(x, width=128):
    parts = x.shape[1] // width
    t = x[:, :width]
    for s in range(1, parts):
        t = t + x[:, s * width:(s + 1) * width]
    return jnp.sum(t, axis=1, keepdims=True)


def _gather_sc(bank, lab0, lab1, lab2, indices):
    B = indices.shape[0]
    D = bank.shape[1]
    NW = 32
    bpw = B // NW
    mesh = plsc.VectorSubcoreMesh(core_axis_name="c", subcore_axis_name="s")

    @functools.partial(
        pl.kernel, mesh=mesh,
        out_type=[jax.ShapeDtypeStruct((B, D), jnp.float32),
                  jax.ShapeDtypeStruct((B,), jnp.int32),
                  jax.ShapeDtypeStruct((B,), jnp.int32),
                  jax.ShapeDtypeStruct((B,), jnp.int32)],
        scratch_types=[pltpu.VMEM((bpw,), jnp.int32),
                       pltpu.VMEM((bpw, D), jnp.float32),
                       pltpu.VMEM((bpw,), jnp.int32),
                       pltpu.SemaphoreType.DMA])
    def k(bank_hbm, l0, l1, l2, idx_hbm, rows_out, b0, b1, b2,
          idx_v, rows_v, lab_v, sem):
        wid = lax.axis_index("s") * 2 + lax.axis_index("c")
        base = wid * bpw
        pltpu.sync_copy(idx_hbm.at[pl.ds(base, bpw)], idx_v)
        pltpu.async_copy(bank_hbm.at[idx_v], rows_v, sem).wait()
        pltpu.sync_copy(rows_v, rows_out.at[pl.ds(base, bpw)])
        for lm, bm in ((l0, b0), (l1, b1), (l2, b2)):
            pltpu.async_copy(lm.at[idx_v], lab_v, sem).wait()
            pltpu.sync_copy(lab_v, bm.at[pl.ds(base, bpw)])

    return k(bank, lab0, lab1, lab2, indices)


def _fused_kernel(nb, kk, padn, nreal, out_ref, bank_ref, labels_ref,
                  blab_ref, s_ref, sc_ref,
                  lo, hi, cl, ch, frac, acc, acc2, sacc, scacc):
    p = pl.program_id(0)
    j = pl.program_id(1)
    B = out_ref.shape[0]
    kf = jnp.float32(kk)
    padf = jnp.float32(padn)

    @pl.when((p == 0) & (j == 0))
    def _init0():
        acc[...] = jnp.zeros_like(acc)
        acc2[...] = jnp.zeros_like(acc2)

    @pl.when((p == 1) & (j == 0))
    def _warm():
        clo = acc[...]
        chi = acc2[...]
        oklo = clo >= kf
        okhi = chi < kf
        lo[...] = jnp.where(oklo, E_LO, -1.01)
        cl[...] = jnp.where(oklo, clo, jnp.float32(nreal))
        hi[...] = jnp.where(okhi, E_HI, 1.01)
        ch[...] = jnp.where(okhi, chi, 0.0)
        acc[...] = jnp.zeros_like(acc)

    @pl.when((p > 1) & (j == 0))
    def _update():
        mid = (lo[...] + hi[...]) * 0.5
        cnt = acc[...] - jnp.where(mid < 0.0, padf, 0.0)
        take = cnt >= kf
        lo[...] = jnp.where(take, mid, lo[...])
        cl[...] = jnp.where(take, cnt, cl[...])
        hi[...] = jnp.where(take, hi[...], mid)
        ch[...] = jnp.where(take, ch[...], cnt)
        acc[...] = jnp.zeros_like(acc)

    @pl.when((p == P + 1) & (j == 0))
    def _frac():
        nrem = kf - ch[...]
        nbr = jnp.maximum(cl[...] - ch[...], 1.0)
        frac[...] = jnp.clip(nrem / nbr, 0.0, 1.0)
        sacc[...] = jnp.zeros_like(sacc)
        scacc[...] = jnp.zeros_like(scacc)

    dp = jax.lax.dot_general(out_ref[...], bank_ref[...],
                             (((1,), (1,)), ((), ())),
                             preferred_element_type=jnp.float32)

    @pl.when(p == 0)
    def _count0():
        acc[...] = acc[...] + _colsum(jnp.where(dp > E_LO, 1.0, 0.0))
        acc2[...] = acc2[...] + _colsum(jnp.where(dp > E_HI, 1.0, 0.0))

    @pl.when((p > 0) & (p <= P))
    def _count():
        mid = (lo[...] + hi[...]) * 0.5
        acc[...] = acc[...] + _colsum(jnp.where(dp > mid, 1.0, 0.0))

    @pl.when(p == P + 1)
    def _final():
        e = jnp.exp(dp * (1.0 / T))
        wgt = jnp.where(dp > hi[...], 1.0,
                        jnp.where(dp > lo[...], frac[...], 0.0))
        we = wgt * e
        lab = labels_ref[0]
        close = ((lab[0:1, :] == blab_ref[:, 0:1])
                 | (lab[1:2, :] == blab_ref[:, 1:2])
                 | (lab[2:3, :] == blab_ref[:, 2:3]))
        sacc[...] = sacc[...] + _colsum(we)
        scacc[...] = scacc[...] + _colsum(jnp.where(close, we, 0.0))

    @pl.when((p == P + 1) & (j == nb - 1))
    def _writeout():
        wgt0 = jnp.where(hi[...] < 0.0, 1.0,
                         jnp.where(lo[...] < 0.0, frac[...], 0.0))
        s_ref[...] = jnp.broadcast_to(sacc[...] - padf * wgt0, s_ref.shape)
        sc_ref[...] = jnp.broadcast_to(scacc[...], sc_ref.shape)


def kernel(indices, outputs, gpu_idx, bank, cluster_labels):
    B, Dm = outputs.shape
    N = bank.shape[0]
    nb = (N + BLKN - 1) // BLKN
    npad = nb * BLKN

    out = _l2n(outputs)
    out_bf = out.astype(jnp.bfloat16)
    bank_bf = jnp.pad(bank, ((0, npad - N), (0, 0))).astype(jnp.bfloat16)

    labels = jnp.pad(cluster_labels, ((0, 0), (0, npad - N)),
                     constant_values=-1)
    labels = labels.reshape(cluster_labels.shape[0], nb, BLKN)
    labels = jnp.transpose(labels, (1, 0, 2))

    rows, b0, b1, b2 = _gather_sc(bank, cluster_labels[0],
                                  cluster_labels[1], cluster_labels[2],
                                  indices)
    blab = jnp.pad(jnp.stack([b0, b1, b2], axis=1), ((0, 0), (0, 5)),
                   constant_values=-2)

    grid = (P + 2, nb)
    s, sc = pl.pallas_call(
        functools.partial(_fused_kernel, nb, K, npad - N, N),
        grid=grid,
        in_specs=[
            pl.BlockSpec((B, Dm), lambda p, j: (0, 0)),
            pl.BlockSpec((BLKN, Dm), lambda p, j: (j, 0)),
            pl.BlockSpec((1, 3, BLKN), lambda p, j: (j, 0, 0)),
            pl.BlockSpec((B, 8), lambda p, j: (0, 0)),
        ],
        out_specs=[
            pl.BlockSpec((B, 128), lambda p, j: (0, 0)),
            pl.BlockSpec((B, 128), lambda p, j: (0, 0)),
        ],
        out_shape=[
            jax.ShapeDtypeStruct((B, 128), jnp.float32),
            jax.ShapeDtypeStruct((B, 128), jnp.float32),
        ],
        scratch_shapes=[
            pltpu.VMEM((B, 1), jnp.float32),
            pltpu.VMEM((B, 1), jnp.float32),
            pltpu.VMEM((B, 1), jnp.float32),
            pltpu.VMEM((B, 1), jnp.float32),
            pltpu.VMEM((B, 1), jnp.float32),
            pltpu.VMEM((B, 1), jnp.float32),
            pltpu.VMEM((B, 1), jnp.float32),
            pltpu.VMEM((B, 1), jnp.float32),
            pltpu.VMEM((B, 1), jnp.float32),
        ],
    )(out_bf, bank_bf, labels, blab)

    s_all = s[:, 0]
    s_close = sc[:, 0]
    loss = -jnp.mean(jnp.log(s_close / s_all + 1e-07))[None]

    new_data_memory = _l2n(rows * M + (1.0 - M) * out)
    return (loss, new_data_memory)
